# Initial kernel scaffold; baseline (speedup 1.0000x reference)
import jax
import jax.numpy as jnp
from jax.experimental import pallas as pl


def kernel(token_ids, weights):
    raise NotImplementedError("write your pallas kernel here")


# speedup vs baseline: 1.1022x; 1.1022x over previous
import functools

import jax
import jax.numpy as jnp
from jax import lax
from jax.experimental import pallas as pl
from jax.experimental.pallas import tpu as pltpu
from jax.experimental.pallas import tpu_sc as plsc

NC = 2
NS = 16
NW = NC * NS


def _emb_lookup(n_total: int, d: int):
    per_w = n_total // NW
    chunk = 1600
    assert per_w % chunk == 0
    n_chunks = per_w // chunk

    @functools.partial(
        pl.kernel,
        mesh=plsc.VectorSubcoreMesh(core_axis_name="c", subcore_axis_name="s"),
        out_type=jax.ShapeDtypeStruct((n_total, d), jnp.float32),
        scratch_types=[
            pltpu.VMEM((chunk,), jnp.int32),
            pltpu.VMEM((chunk, d), jnp.float32),
            pltpu.SemaphoreType.DMA,
        ],
        compiler_params=pltpu.CompilerParams(use_tc_tiling_on_sc=False),
    )
    def body(idx_hbm, table_hbm, out_hbm, idx_v, rows_v, sem):
        wid = lax.axis_index("s") * NC + lax.axis_index("c")
        base = wid * per_w

        def step(i, carry):
            off = base + i * chunk
            pltpu.sync_copy(idx_hbm.at[pl.ds(off, chunk)], idx_v)
            pltpu.async_copy(table_hbm.at[idx_v], rows_v, sem).wait()
            pltpu.sync_copy(rows_v, out_hbm.at[pl.ds(off, chunk)])
            return carry

        lax.fori_loop(0, n_chunks, step, 0)

    return body


def kernel(token_ids, weights):
    b, h = token_ids.shape
    d = weights.shape[1]
    flat = token_ids.reshape(-1).astype(jnp.int32)
    out = _emb_lookup(b * h, d)(flat, weights)
    return out.reshape(b, h, d)


# speedup vs baseline: 1.1115x; 1.0084x over previous
import functools

import jax
import jax.numpy as jnp
from jax import lax
from jax.experimental import pallas as pl
from jax.experimental.pallas import tpu as pltpu
from jax.experimental.pallas import tpu_sc as plsc

NC = 2
NS = 16
NW = NC * NS


def _emb_lookup(n_total: int, d: int):
    per_w = n_total // NW
    chunk = 1600
    assert per_w % (2 * chunk) == 0
    n_pairs = per_w // (2 * chunk)

    @functools.partial(
        pl.kernel,
        mesh=plsc.VectorSubcoreMesh(core_axis_name="c", subcore_axis_name="s"),
        out_type=jax.ShapeDtypeStruct((n_total, d), jnp.float32),
        scratch_types=[
            pltpu.VMEM((per_w,), jnp.int32),
            pltpu.VMEM((chunk, d), jnp.float32),
            pltpu.VMEM((chunk, d), jnp.float32),
            pltpu.SemaphoreType.DMA,
            pltpu.SemaphoreType.DMA,
            pltpu.SemaphoreType.DMA,
            pltpu.SemaphoreType.DMA,
        ],
        compiler_params=pltpu.CompilerParams(use_tc_tiling_on_sc=False),
    )
    def body(idx_hbm, table_hbm, out_hbm, idx_v, rows0, rows1,
             sem_g0, sem_g1, sem_w0, sem_w1):
        wid = lax.axis_index("s") * NC + lax.axis_index("c")
        base = wid * per_w
        pltpu.sync_copy(idx_hbm.at[pl.ds(base, per_w)], idx_v)

        def pair(p, carry):
            o0 = 2 * p * chunk
            o1 = o0 + chunk

            @pl.when(p > 0)
            def _():
                pltpu.make_async_copy(
                    rows0, out_hbm.at[pl.ds(base, chunk)], sem_w0).wait()
                pltpu.make_async_copy(
                    rows1, out_hbm.at[pl.ds(base, chunk)], sem_w1).wait()

            g0 = pltpu.async_copy(
                table_hbm.at[idx_v.at[pl.ds(o0, chunk)]], rows0, sem_g0)
            g1 = pltpu.async_copy(
                table_hbm.at[idx_v.at[pl.ds(o1, chunk)]], rows1, sem_g1)
            g0.wait()
            pltpu.async_copy(rows0, out_hbm.at[pl.ds(base + o0, chunk)], sem_w0)
            g1.wait()
            pltpu.async_copy(rows1, out_hbm.at[pl.ds(base + o1, chunk)], sem_w1)
            return carry

        lax.fori_loop(0, n_pairs, pair, 0)
        pltpu.make_async_copy(
            rows0, out_hbm.at[pl.ds(base, chunk)], sem_w0).wait()
        pltpu.make_async_copy(
            rows1, out_hbm.at[pl.ds(base, chunk)], sem_w1).wait()

    return body


def kernel(token_ids, weights):
    b, h = token_ids.shape
    d = weights.shape[1]
    flat = token_ids.reshape(-1).astype(jnp.int32)
    out = _emb_lookup(b * h, d)(flat, weights)
    return out.reshape(b, h, d)


# speedup vs baseline: 1.6508x; 1.4852x over previous
import functools

import jax
import jax.numpy as jnp
from jax import lax
from jax.experimental import pallas as pl
from jax.experimental.pallas import tpu as pltpu
from jax.experimental.pallas import tpu_sc as plsc

NC = 2
NS = 16
NW = NC * NS
LANE = 128
SUB = 8


def _emb_lookup(b: int, h: int, d: int):
    bpw = b // NW
    nbc = bpw // LANE
    ndr = d // SUB
    ngrid = b // LANE
    assert bpw * NW == b and nbc * LANE == bpw and ndr * SUB == d

    @functools.partial(
        pl.kernel,
        mesh=plsc.VectorSubcoreMesh(core_axis_name="c", subcore_axis_name="s"),
        out_type=jax.ShapeDtypeStruct((h * ndr * ngrid, SUB * LANE),
                                      jnp.float32),
        scratch_types=[
            pltpu.VMEM((2, bpw), jnp.int32),
            pltpu.VMEM((2, bpw, d), jnp.float32),
            pltpu.VMEM((2, ndr, nbc, SUB * LANE), jnp.float32),
            pltpu.SemaphoreType.DMA,
            pltpu.SemaphoreType.DMA,
            pltpu.SemaphoreType.DMA,
            pltpu.SemaphoreType.DMA,
        ],
        compiler_params=pltpu.CompilerParams(
            use_tc_tiling_on_sc=False, needs_layout_passes=False),
    )
    def body(idx_hbm, table_hbm, out_hbm, idx_v, rows_v, obuf,
             sem_i, sem_g, sem_w0, sem_w1):
        wid = lax.axis_index("s") * NC + lax.axis_index("c")
        b0 = wid * bpw
        bc0 = wid * nbc
        iota16 = jax.lax.iota(jnp.int32, 16)

        def stage(hh, slot):
            pltpu.async_copy(
                idx_hbm.at[pl.ds(hh * b + b0, bpw)], idx_v.at[slot], sem_i)

        def gather(slot):
            pltpu.async_copy(
                table_hbm.at[idx_v.at[slot]], rows_v.at[slot], sem_g)

        def wait_idx(slot):
            pltpu.make_async_copy(
                idx_hbm.at[pl.ds(b0, bpw)], idx_v.at[slot], sem_i).wait()

        def wait_gather(slot):
            pltpu.make_async_copy(
                table_hbm.at[idx_v.at[slot]], rows_v.at[slot], sem_g).wait()

        def fire_out(hh, slot, sem):
            for dr in range(ndr):
                pltpu.async_copy(
                    obuf.at[slot, dr],
                    out_hbm.at[pl.ds((hh * ndr + dr) * ngrid + bc0, nbc)],
                    sem)

        def wait_out(slot, sem):
            for dr in range(ndr):
                pltpu.make_async_copy(
                    obuf.at[slot, dr],
                    out_hbm.at[pl.ds(bc0, nbc)], sem).wait()

        stage(0, 0)
        wait_idx(0)
        gather(0)
        stage(1, 1)

        def h_body(hh, carry):
            slot = lax.rem(hh, 2)

            @pl.when(hh + 1 < h)
            def _():
                wait_idx(1 - slot)
            wait_gather(slot)

            @pl.when(hh + 1 < h)
            def _():
                gather(1 - slot)

            @pl.when(hh + 2 < h)
            def _():
                stage(hh + 2, slot)

            @pl.when((hh >= 2) & (slot == 0))
            def _():
                wait_out(0, sem_w0)

            @pl.when((hh >= 2) & (slot == 1))
            def _():
                wait_out(1, sem_w1)

            rows = rows_v.at[slot]

            def t_body(r, carry2):
                dr = r // (nbc * SUB)
                rem = lax.rem(r, nbc * SUB)
                bc = rem // SUB
                s = lax.rem(rem, SUB)
                col = jnp.broadcast_to(dr * SUB + s, (16,)).astype(jnp.int32)
                dst = obuf.at[slot, dr, bc]

                def k_body(k, carry3):
                    row_idx = bc * LANE + k * 16 + iota16
                    v = plsc.load_gather(rows, [row_idx, col])
                    dst[pl.ds(s * LANE + k * 16, 16)] = v
                    return carry3

                lax.fori_loop(0, LANE // 16, k_body, 0, unroll=True)
                return carry2

            lax.fori_loop(0, ndr * nbc * SUB, t_body, 0, unroll=2)

            @pl.when(slot == 0)
            def _():
                fire_out(hh, 0, sem_w0)

            @pl.when(slot == 1)
            def _():
                fire_out(hh, 1, sem_w1)

            return carry

        lax.fori_loop(0, h, h_body, 0)

        wait_out(0, sem_w0)
        wait_out(1, sem_w1)

    return body


def kernel(token_ids, weights):
    b, h = token_ids.shape
    d = weights.shape[1]
    flat_t = token_ids.T.reshape(-1).astype(jnp.int32)
    out2 = _emb_lookup(b, h, d)(flat_t, weights)
    out5 = out2.reshape(h, d // SUB, b // LANE, SUB, LANE)
    return out5.transpose(2, 4, 0, 1, 3).reshape(b, h, d)


# speedup vs baseline: 1.8703x; 1.1330x over previous
import functools

import jax
import jax.numpy as jnp
from jax import lax
from jax.experimental import pallas as pl
from jax.experimental.pallas import tpu as pltpu
from jax.experimental.pallas import tpu_sc as plsc

NC = 2
NS = 16
NW = NC * NS
LANE = 128
SUB = 8


def _emb_lookup(b: int, h: int, d: int):
    bpw = b // NW
    nbc = bpw // LANE
    ndr = d // SUB
    ngrid = b // LANE
    assert bpw * NW == b and nbc * LANE == bpw and ndr * SUB == d

    @functools.partial(
        pl.kernel,
        mesh=plsc.VectorSubcoreMesh(core_axis_name="c", subcore_axis_name="s"),
        out_type=jax.ShapeDtypeStruct((h * ndr * ngrid * SUB * LANE,),
                                      jnp.float32),
        scratch_types=[
            pltpu.VMEM((2, bpw), jnp.int32),
            pltpu.VMEM((2, bpw, d), jnp.float32),
            pltpu.VMEM((2, ndr * nbc * SUB * LANE), jnp.float32),
            pltpu.SemaphoreType.DMA,
            pltpu.SemaphoreType.DMA,
            pltpu.SemaphoreType.DMA,
            pltpu.SemaphoreType.DMA,
        ],
        compiler_params=pltpu.CompilerParams(
            use_tc_tiling_on_sc=False, needs_layout_passes=False),
    )
    def body(idx_hbm, table_hbm, out_hbm, idx_v, rows_v, obuf,
             sem_i, sem_g, sem_w0, sem_w1):
        wid = lax.axis_index("s") * NC + lax.axis_index("c")
        b0 = wid * bpw
        bc0 = wid * nbc
        iota16 = jax.lax.iota(jnp.int32, 16)
        perm_lo = (iota16 // SUB) * (nbc * SUB * LANE) + (iota16 % SUB) * LANE
        perm_hi = perm_lo + 2 * (nbc * SUB * LANE)

        def stage(hh, slot):
            pltpu.async_copy(
                idx_hbm.at[pl.ds(hh * b + b0, bpw)], idx_v.at[slot], sem_i)

        def gather(slot):
            pltpu.async_copy(
                table_hbm.at[idx_v.at[slot]], rows_v.at[slot], sem_g)

        def wait_idx(slot):
            pltpu.make_async_copy(
                idx_hbm.at[pl.ds(b0, bpw)], idx_v.at[slot], sem_i).wait()

        def wait_gather(slot):
            pltpu.make_async_copy(
                table_hbm.at[idx_v.at[slot]], rows_v.at[slot], sem_g).wait()

        def fire_out(hh, slot, sem):
            for dr in range(ndr):
                pltpu.async_copy(
                    obuf.at[slot, pl.ds(dr * nbc * SUB * LANE, nbc * SUB * LANE)],
                    out_hbm.at[pl.ds(
                        ((hh * ndr + dr) * ngrid + bc0) * SUB * LANE,
                        nbc * SUB * LANE)],
                    sem)

        def wait_out(slot, sem):
            for dr in range(ndr):
                pltpu.make_async_copy(
                    obuf.at[slot, pl.ds(0, nbc * SUB * LANE)],
                    out_hbm.at[pl.ds(0, nbc * SUB * LANE)], sem).wait()

        stage(0, 0)
        wait_idx(0)
        gather(0)
        stage(1, 1)

        def h_body(hh, carry):
            slot = lax.rem(hh, 2)

            @pl.when(hh + 1 < h)
            def _():
                wait_idx(1 - slot)
            wait_gather(slot)

            @pl.when(hh + 1 < h)
            def _():
                gather(1 - slot)

            @pl.when(hh + 2 < h)
            def _():
                stage(hh + 2, slot)

            @pl.when((hh >= 2) & (slot == 0))
            def _():
                wait_out(0, sem_w0)

            @pl.when((hh >= 2) & (slot == 1))
            def _():
                wait_out(1, sem_w1)

            rows = rows_v.at[slot]
            odst = obuf.at[slot]

            def t_body(j, carry2):
                cj = (j // LANE) * (SUB * LANE) + lax.rem(j, LANE)
                v0 = rows[j, pl.ds(0, 16)]
                v1 = rows[j, pl.ds(16, 16)]
                plsc.store_scatter(odst, [perm_lo + cj], v0)
                plsc.store_scatter(odst, [perm_hi + cj], v1)
                return carry2

            lax.fori_loop(0, bpw, t_body, 0, unroll=8)

            @pl.when(slot == 0)
            def _():
                fire_out(hh, 0, sem_w0)

            @pl.when(slot == 1)
            def _():
                fire_out(hh, 1, sem_w1)

            return carry

        lax.fori_loop(0, h, h_body, 0)

        wait_out(0, sem_w0)
        wait_out(1, sem_w1)

    return body


def kernel(token_ids, weights):
    b, h = token_ids.shape
    d = weights.shape[1]
    flat_t = token_ids.T.reshape(-1).astype(jnp.int32)
    out1 = _emb_lookup(b, h, d)(flat_t, weights)
    out5 = out1.reshape(h, d // SUB, b // LANE, SUB, LANE)
    return out5.transpose(2, 4, 0, 1, 3).reshape(b, h, d)
